# baseline (speedup 1.0000x reference)
import functools

import jax
import jax.numpy as jnp
from jax import lax
from jax.experimental import pallas as pl
from jax.experimental.pallas import tpu as pltpu
from jax.experimental.pallas import tpu_sc as plsc

_B, _T, _S, _H = 16, 4, 1025, 1280
_CH = 8
_NCHS = 1024 // _CH
_NSL = 4
_NCH = _NSL * _NCHS
_RING = 4
_TW = 128


def _sc_body(hs_ref, jrows_ref, table_ref, out_ref,
             jidx_v, emb_v, in_buf, out_buf, gsem, in_sem, out_sem):
    wid = lax.axis_index("s")

    pltpu.sync_copy(jrows_ref.at[wid], jidx_v)
    pltpu.make_async_copy(table_ref.at[jidx_v], emb_v, gsem).start()
    pltpu.make_async_copy(table_ref.at[jidx_v], emb_v, gsem).wait()

    def chunk_addr(k):
        q = k // _NCHS
        l = lax.rem(k, _NCHS)
        bt = _NSL * wid + q
        return bt // _T, lax.rem(bt, _T), pl.ds(l * _CH, _CH)

    def in_copy(k, bi):
        b, t, rows = chunk_addr(k)
        return pltpu.make_async_copy(hs_ref.at[b, t, rows], in_buf.at[bi], in_sem.at[bi])

    def out_copy(k, bi):
        b, t, rows = chunk_addr(k)
        return pltpu.make_async_copy(out_buf.at[bi], out_ref.at[b, t, rows], out_sem.at[bi])

    for i in range(_RING):
        in_copy(i, i).start()

    def g_body(g, carry):
        for i in range(_RING):
            k = _RING * g + i
            in_copy(k, i).wait()

            @pl.when(g >= 1)
            def _reclaim_out():
                out_copy(k - _RING, i).wait()

            q = k // _NCHS
            for h in range(2):
                e = [emb_v[10 * q + (40 * h + ci) // 8,
                           pl.ds((lax.rem(40 * h + ci, 8)) * 16, 16)]
                     for ci in range(40)]

                def row_body(r, c2, _h=h, _e=e, _i=i):
                    for ci in range(40):
                        c = 40 * _h + ci
                        out_buf[_i, r, pl.ds(c * 16, 16)] = (
                            in_buf[_i, r, pl.ds(c * 16, 16)] + _e[ci])
                    return c2

                lax.fori_loop(0, _CH, row_body, 0)

            out_copy(k, i).start()

            @pl.when(g <= _NCH // _RING - 2)
            def _prefetch():
                in_copy(k + _RING, i).start()
        return carry

    lax.fori_loop(0, _NCH // _RING, g_body, 0)

    for i in range(_RING):
        out_copy(_NCH - _RING + i, i).wait()


def _last_row_body(ids_ref, hs_ref, emb_ref, prev_ref, out_ref):
    del ids_ref, prev_ref
    out_ref[...] = hs_ref[...] + emb_ref[...]


def kernel(hidden_states, aspect_ratio_ids, embedding_weight):
    ids = aspect_ratio_ids.astype(jnp.int32)
    bt = jnp.arange(_B * _T, dtype=jnp.int32)
    j64 = ids[bt // _T] * _T + bt % _T
    jrows = (j64[:, None] * 10 + jnp.arange(10, dtype=jnp.int32)[None, :]).reshape(16, 40)
    table = embedding_weight.reshape(-1, _TW)

    mesh = plsc.VectorSubcoreMesh(core_axis_name="c", subcore_axis_name="s",
                                  num_cores=1)
    sc_add = functools.partial(
        pl.kernel,
        out_type=jax.ShapeDtypeStruct((_B, _T, _S, _H), jnp.float32),
        mesh=mesh,
        scratch_types=[
            pltpu.VMEM((40,), jnp.int32),
            pltpu.VMEM((40, _TW), jnp.float32),
            pltpu.VMEM((_RING, _CH, _H), jnp.float32),
            pltpu.VMEM((_RING, _CH, _H), jnp.float32),
            pltpu.SemaphoreType.DMA,
            pltpu.SemaphoreType.DMA((_RING,)),
            pltpu.SemaphoreType.DMA((_RING,)),
        ],
    )(_sc_body)
    out = sc_add(hidden_states, jrows, table)

    table3 = embedding_weight.reshape(-1, 1, _H)
    _LB = _S // 8
    return pl.pallas_call(
        _last_row_body,
        grid_spec=pltpu.PrefetchScalarGridSpec(
            num_scalar_prefetch=1,
            grid=(_B, _T),
            in_specs=[
                pl.BlockSpec((1, 1, 8, _H), lambda b, t, ids_ref: (b, t, _LB, 0)),
                pl.BlockSpec((1, 1, _H),
                             lambda b, t, ids_ref: (ids_ref[b] * _T + t, 0, 0)),
                pl.BlockSpec(memory_space=pl.ANY),
            ],
            out_specs=pl.BlockSpec((1, 1, 8, _H), lambda b, t, ids_ref: (b, t, _LB, 0)),
        ),
        out_shape=jax.ShapeDtypeStruct((_B, _T, _S, _H), jnp.float32),
        input_output_aliases={3: 0},
    )(ids, hidden_states, table3, out)


# speedup vs baseline: 1.1244x; 1.1244x over previous
import functools

import jax
import jax.numpy as jnp
from jax import lax
from jax.experimental import pallas as pl
from jax.experimental.pallas import tpu as pltpu
from jax.experimental.pallas import tpu_sc as plsc

_B, _T, _S, _H = 16, 4, 1025, 1280
_CH = 8
_NCHS = 1024 // _CH
_NCH = 2 * _NCHS
_RING = 4
_TW = 128


def _sc_body(hs_ref, jrows_ref, table_ref, out_ref,
             jidx_v, emb_v, in_buf, out_buf, gsem, in_sem, out_sem):
    wid = lax.axis_index("s") * 2 + lax.axis_index("c")

    pltpu.sync_copy(jrows_ref.at[wid], jidx_v)
    pltpu.make_async_copy(table_ref.at[jidx_v], emb_v, gsem).start()
    pltpu.make_async_copy(table_ref.at[jidx_v], emb_v, gsem).wait()

    def chunk_addr(k):
        q = k // _NCHS
        l = lax.rem(k, _NCHS)
        bt = 2 * wid + q
        return bt // _T, lax.rem(bt, _T), pl.ds(l * _CH, _CH)

    def in_copy(k, bi):
        b, t, rows = chunk_addr(k)
        return pltpu.make_async_copy(hs_ref.at[b, t, rows], in_buf.at[bi], in_sem.at[bi])

    def out_copy(k, bi):
        b, t, rows = chunk_addr(k)
        return pltpu.make_async_copy(out_buf.at[bi], out_ref.at[b, t, rows], out_sem.at[bi])

    for i in range(_RING):
        in_copy(i, i).start()

    def g_body(g, carry):
        for i in range(_RING):
            k = _RING * g + i
            in_copy(k, i).wait()

            @pl.when(g >= 1)
            def _reclaim_out():
                out_copy(k - _RING, i).wait()

            q = k // _NCHS
            for h in range(2):
                e = [emb_v[10 * q + (40 * h + ci) // 8,
                           pl.ds((lax.rem(40 * h + ci, 8)) * 16, 16)]
                     for ci in range(40)]

                def row_body(r, c2, _h=h, _e=e, _i=i):
                    for ci in range(40):
                        c = 40 * _h + ci
                        out_buf[_i, r, pl.ds(c * 16, 16)] = (
                            in_buf[_i, r, pl.ds(c * 16, 16)] + _e[ci])
                    return c2

                lax.fori_loop(0, _CH, row_body, 0)

            out_copy(k, i).start()

            @pl.when(g <= _NCH // _RING - 2)
            def _prefetch():
                in_copy(k + _RING, i).start()
        return carry

    lax.fori_loop(0, _NCH // _RING, g_body, 0)

    for i in range(_RING):
        out_copy(_NCH - _RING + i, i).wait()


def _last_row_body(ids_ref, hs_ref, table_ref, prev_ref, out_ref):
    del prev_ref
    for b in range(_B):
        for t in range(_T):
            j = ids_ref[b] * _T + t
            out_ref[b, t] = hs_ref[b, t] + table_ref[pl.ds(j, 1), :]


def kernel(hidden_states, aspect_ratio_ids, embedding_weight):
    ids = aspect_ratio_ids.astype(jnp.int32)
    bt = jnp.arange(_B * _T, dtype=jnp.int32)
    j64 = ids[bt // _T] * _T + bt % _T
    jr = (j64[:, None] * 10 + jnp.arange(10, dtype=jnp.int32)[None, :]).reshape(32, 20)
    jrows = jnp.concatenate([jr, jr[:, -4:]], axis=1)
    table = embedding_weight.reshape(-1, _TW)

    mesh = plsc.VectorSubcoreMesh(core_axis_name="c", subcore_axis_name="s")
    sc_add = functools.partial(
        pl.kernel,
        out_type=jax.ShapeDtypeStruct((_B, _T, _S, _H), jnp.float32),
        mesh=mesh,
        scratch_types=[
            pltpu.VMEM((24,), jnp.int32),
            pltpu.VMEM((24, _TW), jnp.float32),
            pltpu.VMEM((_RING, _CH, _H), jnp.float32),
            pltpu.VMEM((_RING, _CH, _H), jnp.float32),
            pltpu.SemaphoreType.DMA,
            pltpu.SemaphoreType.DMA((_RING,)),
            pltpu.SemaphoreType.DMA((_RING,)),
        ],
    )(_sc_body)
    out = sc_add(hidden_states, jrows, table)

    table2 = embedding_weight.reshape(-1, _H)
    _LB = _S // 8
    return pl.pallas_call(
        _last_row_body,
        grid_spec=pltpu.PrefetchScalarGridSpec(
            num_scalar_prefetch=1,
            grid=(1,),
            in_specs=[
                pl.BlockSpec((_B, _T, 8, _H), lambda i, ids_ref: (0, 0, _LB, 0)),
                pl.BlockSpec((table2.shape[0], _H), lambda i, ids_ref: (0, 0)),
                pl.BlockSpec(memory_space=pl.ANY),
            ],
            out_specs=pl.BlockSpec((_B, _T, 8, _H), lambda i, ids_ref: (0, 0, _LB, 0)),
        ),
        out_shape=jax.ShapeDtypeStruct((_B, _T, _S, _H), jnp.float32),
        input_output_aliases={3: 0},
    )(ids, hidden_states, table2, out)
